# baseline (speedup 1.0000x reference)
import jax
import jax.numpy as jnp
from jax.experimental import pallas as pl
from jax.experimental.pallas import tpu as pltpu


def _copy_body(src_ref, dst_ref, sem):
    cp = pltpu.make_async_copy(src_ref, dst_ref, sem)
    cp.start()
    cp.wait()


def kernel(embeddings):
    return pl.pallas_call(
        _copy_body,
        out_shape=jax.ShapeDtypeStruct(embeddings.shape, embeddings.dtype),
        in_specs=[pl.BlockSpec(memory_space=pl.ANY)],
        out_specs=pl.BlockSpec(memory_space=pl.ANY),
        scratch_shapes=[pltpu.SemaphoreType.DMA],
    )(embeddings)


# speedup vs baseline: 1.0004x; 1.0004x over previous
import jax
import jax.numpy as jnp
from jax.experimental import pallas as pl
from jax.experimental.pallas import tpu as pltpu


_NUM_CHUNKS = 8


def _copy_body(src_ref, dst_ref, sems):
    rows = src_ref.shape[0]
    chunk = rows // _NUM_CHUNKS
    copies = [
        pltpu.make_async_copy(
            src_ref.at[pl.ds(i * chunk, chunk)],
            dst_ref.at[pl.ds(i * chunk, chunk)],
            sems.at[i],
        )
        for i in range(_NUM_CHUNKS)
    ]
    for cp in copies:
        cp.start()
    for cp in copies:
        cp.wait()


def kernel(embeddings):
    return pl.pallas_call(
        _copy_body,
        out_shape=jax.ShapeDtypeStruct(embeddings.shape, embeddings.dtype),
        in_specs=[pl.BlockSpec(memory_space=pl.ANY)],
        out_specs=pl.BlockSpec(memory_space=pl.ANY),
        scratch_shapes=[pltpu.SemaphoreType.DMA((_NUM_CHUNKS,))],
    )(embeddings)


# speedup vs baseline: 16.1405x; 16.1333x over previous
import jax
import jax.numpy as jnp
from jax.experimental import pallas as pl
from jax.experimental.pallas import tpu as pltpu


_BLOCK_ROWS = 8000


def _copy_body(src_ref, dst_ref):
    dst_ref[...] = src_ref[...]


def kernel(embeddings):
    rows, dim = embeddings.shape
    grid = rows // _BLOCK_ROWS
    return pl.pallas_call(
        _copy_body,
        out_shape=jax.ShapeDtypeStruct(embeddings.shape, embeddings.dtype),
        grid=(grid,),
        in_specs=[pl.BlockSpec((_BLOCK_ROWS, dim), lambda i: (i, 0))],
        out_specs=pl.BlockSpec((_BLOCK_ROWS, dim), lambda i: (i, 0)),
    )(embeddings)
